# Initial kernel scaffold; baseline (speedup 1.0000x reference)
import jax
import jax.numpy as jnp
from jax.experimental import pallas as pl


def kernel(Z, table):
    raise NotImplementedError("write your pallas kernel here")


# speedup vs baseline: 1.7589x; 1.7589x over previous
import functools

import jax
import jax.numpy as jnp
from jax import lax
from jax.experimental import pallas as pl
from jax.experimental.pallas import tpu as pltpu
from jax.experimental.pallas import tpu_sc as plsc

EMB = 128
N_ROWS = 100000
CHUNK = 800
NCHUNKS = N_ROWS // CHUNK
NUM_WORKERS = 32
ITERS = -(-NCHUNKS // NUM_WORKERS)


@jax.jit
def _sc_gather(table94, idx):
    mesh = plsc.VectorSubcoreMesh(core_axis_name="c", subcore_axis_name="s")

    @functools.partial(
        pl.kernel,
        out_type=jax.ShapeDtypeStruct((N_ROWS, EMB), jnp.float32),
        mesh=mesh,
        scratch_types=[
            pltpu.VMEM((CHUNK,), jnp.int32),
            pltpu.VMEM((CHUNK, EMB), jnp.float32),
            pltpu.SemaphoreType.DMA,
        ],
    )
    def k(table_hbm, idx_hbm, out_hbm, idx_v, rows_v, sem):
        w = lax.axis_index("s") * 2 + lax.axis_index("c")
        for i in range(ITERS):
            c = w + NUM_WORKERS * i

            @pl.when(c < NCHUNKS)
            def _():
                base = c * CHUNK
                pltpu.sync_copy(idx_hbm.at[pl.ds(base, CHUNK)], idx_v)
                pltpu.async_copy(table_hbm.at[idx_v], rows_v, sem).wait()
                pltpu.sync_copy(rows_v, out_hbm.at[pl.ds(base, CHUNK)])

    return k(table94, idx)


def kernel(Z, table):
    table94 = jnp.concatenate([jnp.zeros((1, EMB), table.dtype), table], axis=0)
    return _sc_gather(table94, Z.astype(jnp.int32))


# speedup vs baseline: 1.7766x; 1.0101x over previous
import functools

import jax
import jax.numpy as jnp
from jax import lax
from jax.experimental import pallas as pl
from jax.experimental.pallas import tpu as pltpu
from jax.experimental.pallas import tpu_sc as plsc

EMB = 128
N_ROWS = 100000
CHUNK = 400
NCHUNKS = N_ROWS // CHUNK
NUM_WORKERS = 32
ITERS = -(-NCHUNKS // NUM_WORKERS)
NBUF = 2


@jax.jit
def _sc_gather(table94, idx):
    mesh = plsc.VectorSubcoreMesh(core_axis_name="c", subcore_axis_name="s")

    @functools.partial(
        pl.kernel,
        out_type=jax.ShapeDtypeStruct((N_ROWS, EMB), jnp.float32),
        mesh=mesh,
        scratch_types=(
            [pltpu.VMEM((CHUNK,), jnp.int32) for _ in range(NBUF)]
            + [pltpu.VMEM((CHUNK, EMB), jnp.float32) for _ in range(NBUF)]
            + [pltpu.SemaphoreType.DMA for _ in range(2 * NBUF)]
        ),
    )
    def k(table_hbm, idx_hbm, out_hbm, *scratch):
        idx_b = scratch[:NBUF]
        row_b = scratch[NBUF:2 * NBUF]
        gsem = scratch[2 * NBUF:3 * NBUF]
        wsem = scratch[3 * NBUF:]
        w = lax.axis_index("s") * 2 + lax.axis_index("c")

        def start_gather(i):
            b = i % NBUF
            c = w + NUM_WORKERS * i

            @pl.when(c < NCHUNKS)
            def _():
                pltpu.sync_copy(idx_hbm.at[pl.ds(c * CHUNK, CHUNK)], idx_b[b])
                pltpu.async_copy(table_hbm.at[idx_b[b]], row_b[b], gsem[b])

        def start_write(i):
            b = i % NBUF
            c = w + NUM_WORKERS * i

            @pl.when(c < NCHUNKS)
            def _():
                pltpu.make_async_copy(table_hbm.at[idx_b[b]], row_b[b],
                                      gsem[b]).wait()
                pltpu.async_copy(row_b[b], out_hbm.at[pl.ds(c * CHUNK, CHUNK)],
                                 wsem[b])

        def finish_write(i):
            b = i % NBUF
            c = w + NUM_WORKERS * i

            @pl.when(c < NCHUNKS)
            def _():
                pltpu.make_async_copy(row_b[b],
                                      out_hbm.at[pl.ds(c * CHUNK, CHUNK)],
                                      wsem[b]).wait()

        for i in range(NBUF):
            start_gather(i)
        for i in range(ITERS):
            start_write(i)
            if i + NBUF < ITERS:
                finish_write(i)
                start_gather(i + NBUF)
        for i in range(ITERS - NBUF, ITERS):
            finish_write(i)

    return k(table94, idx)


def kernel(Z, table):
    table94 = jnp.concatenate([jnp.zeros((1, EMB), table.dtype), table], axis=0)
    return _sc_gather(table94, Z.astype(jnp.int32))


# speedup vs baseline: 3.2573x; 1.8334x over previous
import functools

import jax
import jax.numpy as jnp
from jax import lax
from jax.experimental import pallas as pl
from jax.experimental.pallas import tpu as pltpu
from jax.experimental.pallas import tpu_sc as plsc

EMB = 128
N_ROWS = 100000
CHUNK = 400
NCHUNKS = N_ROWS // CHUNK
NUM_WORKERS = 32
ITERS = -(-NCHUNKS // NUM_WORKERS)
NBUF = 2
NREP = 32
TROWS = 94 * NREP


@jax.jit
def _sc_gather(table94, idx):
    mesh = plsc.VectorSubcoreMesh(core_axis_name="c", subcore_axis_name="s")

    @functools.partial(
        pl.kernel,
        out_type=jax.ShapeDtypeStruct((N_ROWS, EMB), jnp.float32),
        mesh=mesh,
        scratch_types=(
            [pltpu.VMEM((CHUNK,), jnp.int32) for _ in range(NBUF)]
            + [pltpu.VMEM((CHUNK, EMB), jnp.float32) for _ in range(NBUF)]
            + [pltpu.SemaphoreType.DMA for _ in range(2 * NBUF)]
        ),
    )
    def k(table_hbm, idx_hbm, out_hbm, *scratch):
        idx_b = scratch[:NBUF]
        row_b = scratch[NBUF:2 * NBUF]
        gsem = scratch[2 * NBUF:3 * NBUF]
        wsem = scratch[3 * NBUF:]
        w = lax.axis_index("s") * 2 + lax.axis_index("c")

        def start_gather(i):
            b = i % NBUF
            c = w + NUM_WORKERS * i

            @pl.when(c < NCHUNKS)
            def _():
                pltpu.sync_copy(idx_hbm.at[pl.ds(c * CHUNK, CHUNK)], idx_b[b])
                pltpu.async_copy(table_hbm.at[idx_b[b]], row_b[b], gsem[b])

        def start_write(i):
            b = i % NBUF
            c = w + NUM_WORKERS * i

            @pl.when(c < NCHUNKS)
            def _():
                pltpu.make_async_copy(table_hbm.at[idx_b[b]], row_b[b],
                                      gsem[b]).wait()
                pltpu.async_copy(row_b[b], out_hbm.at[pl.ds(c * CHUNK, CHUNK)],
                                 wsem[b])

        def finish_write(i):
            b = i % NBUF
            c = w + NUM_WORKERS * i

            @pl.when(c < NCHUNKS)
            def _():
                pltpu.make_async_copy(row_b[b],
                                      out_hbm.at[pl.ds(c * CHUNK, CHUNK)],
                                      wsem[b]).wait()

        for i in range(NBUF):
            start_gather(i)
        for i in range(ITERS):
            start_write(i)
            if i + NBUF < ITERS:
                finish_write(i)
                start_gather(i + NBUF)
        for i in range(ITERS - NBUF, ITERS):
            finish_write(i)

    return k(table94, idx)


def kernel(Z, table):
    table94 = jnp.concatenate([jnp.zeros((1, EMB), table.dtype), table], axis=0)
    table_rep = jnp.tile(table94, (NREP, 1))
    rep = (jnp.arange(N_ROWS, dtype=jnp.int32) // CHUNK) % NREP
    idx = Z.astype(jnp.int32) + 94 * rep
    return _sc_gather(table_rep, idx)


# speedup vs baseline: 5.2917x; 1.6245x over previous
import functools

import jax
import jax.numpy as jnp
from jax import lax
from jax.experimental import pallas as pl
from jax.experimental.pallas import tpu as pltpu
from jax.experimental.pallas import tpu_sc as plsc

EMB = 128
N_ROWS = 100000
TROWS = 94
CHUNK = 400
NCHUNKS = N_ROWS // CHUNK
NUM_WORKERS = 32
ITERS = -(-NCHUNKS // NUM_WORKERS)
NBUF = 2


@jax.jit
def _sc_gather(table94, idx):
    mesh = plsc.VectorSubcoreMesh(core_axis_name="c", subcore_axis_name="s")

    @functools.partial(
        pl.kernel,
        out_type=jax.ShapeDtypeStruct((N_ROWS, EMB), jnp.float32),
        mesh=mesh,
        scratch_types=(
            [pltpu.VMEM_SHARED((TROWS, EMB), jnp.float32)]
            + [pltpu.VMEM((CHUNK,), jnp.int32) for _ in range(NBUF)]
            + [pltpu.VMEM((CHUNK, EMB), jnp.float32) for _ in range(NBUF)]
            + [pltpu.SemaphoreType.DMA for _ in range(2 * NBUF)]
        ),
    )
    def k(table_hbm, idx_hbm, out_hbm, table_v, *scratch):
        idx_b = scratch[:NBUF]
        row_b = scratch[NBUF:2 * NBUF]
        gsem = scratch[2 * NBUF:3 * NBUF]
        wsem = scratch[3 * NBUF:]
        w = lax.axis_index("s") * 2 + lax.axis_index("c")

        @pl.when(lax.axis_index("s") == 0)
        def _():
            pltpu.sync_copy(table_hbm, table_v)

        plsc.subcore_barrier()

        def start_gather(i):
            b = i % NBUF
            c = w + NUM_WORKERS * i

            @pl.when(c < NCHUNKS)
            def _():
                pltpu.sync_copy(idx_hbm.at[pl.ds(c * CHUNK, CHUNK)], idx_b[b])
                pltpu.async_copy(table_v.at[idx_b[b]], row_b[b], gsem[b])

        def start_write(i):
            b = i % NBUF
            c = w + NUM_WORKERS * i

            @pl.when(c < NCHUNKS)
            def _():
                pltpu.make_async_copy(table_v.at[idx_b[b]], row_b[b],
                                      gsem[b]).wait()
                pltpu.async_copy(row_b[b], out_hbm.at[pl.ds(c * CHUNK, CHUNK)],
                                 wsem[b])

        def finish_write(i):
            b = i % NBUF
            c = w + NUM_WORKERS * i

            @pl.when(c < NCHUNKS)
            def _():
                pltpu.make_async_copy(row_b[b],
                                      out_hbm.at[pl.ds(c * CHUNK, CHUNK)],
                                      wsem[b]).wait()

        for i in range(NBUF):
            start_gather(i)
        for i in range(ITERS):
            start_write(i)
            if i + NBUF < ITERS:
                finish_write(i)
                start_gather(i + NBUF)
        for i in range(ITERS - NBUF, ITERS):
            finish_write(i)

    return k(table94, idx)


def kernel(Z, table):
    table94 = jnp.concatenate([jnp.zeros((1, EMB), table.dtype), table], axis=0)
    return _sc_gather(table94, Z.astype(jnp.int32))
